# baseline (speedup 1.0000x reference)
import jax
import jax.numpy as jnp
from jax.experimental import pallas as pl
from jax.experimental.pallas import tpu as pltpu
from jax.experimental.pallas import tpu_sc as plsc

B, T, V, D = 4096, 8, 1024, 256
NCHUNK = 4
BCH = B // NCHUNK
BB = 512
BC = 512
W = 128


def _argmax_block(msg_ref, idx_ref):
    m = msg_ref[...]
    mx = jnp.max(m, axis=-1, keepdims=True)
    iota3 = jax.lax.broadcasted_iota(jnp.int32, (BB, T, V), 2)
    codes = jnp.min(jnp.where(m == mx, iota3, V), axis=-1)
    level = jax.lax.broadcasted_iota(jnp.int32, (BB, T), 1)
    idx_ref[...] = codes + V * level


def _cumnorm_block(g_ref, out_ref):
    g = g_ref[...]
    c = g
    for k in (1, 2, 4):
        c = c + jnp.pad(c, ((0, 0), (k, 0), (0, 0)))[:, :T, :]
    norm = jnp.sqrt(jnp.sum(c * c, axis=-1, keepdims=True))
    out_ref[...] = c * (1.0 / jnp.maximum(norm, 1e-12))


def _sc_gather(table, idx_flat):
    mesh = plsc.VectorSubcoreMesh(core_axis_name="core", subcore_axis_name="subcore")

    @pl.kernel(
        out_type=jax.ShapeDtypeStruct((BCH * T, D), jnp.float32),
        mesh=mesh,
    )
    def gather_kernel(x_hbm, i_hbm, o_hbm):
        def body(i_vmem, o_vmem):
            pltpu.sync_copy(x_hbm.at[i_vmem.at[0]], o_vmem)

        pltpu.emit_pipeline(
            body,
            grid=(BCH * T // W,),
            in_specs=[pl.BlockSpec((1, W), index_map=lambda i: (0, i))],
            out_specs=[pl.BlockSpec((W, D), index_map=lambda i: (i, 0))],
            core_axis_name="subcore",
            dimension_semantics=(pltpu.PARALLEL,),
        )(i_hbm, o_hbm)

    return gather_kernel(table, idx_flat)


@jax.jit
def kernel(message, codebooks):
    table = codebooks.reshape(T * V, D)
    outs = []
    for ci in range(NCHUNK):
        msg_c = jax.lax.slice_in_dim(message, ci * BCH, (ci + 1) * BCH, axis=0)
        idx = pl.pallas_call(
            _argmax_block,
            grid=(BCH // BB,),
            in_specs=[pl.BlockSpec((BB, T, V), lambda i: (i, 0, 0))],
            out_specs=pl.BlockSpec((BB, T), lambda i: (i, 0)),
            out_shape=jax.ShapeDtypeStruct((BCH, T), jnp.int32),
        )(msg_c)
        gathered = _sc_gather(table, idx.reshape(1, BCH * T))
        out_c = pl.pallas_call(
            _cumnorm_block,
            grid=(BCH // BC,),
            in_specs=[pl.BlockSpec((BC, T, D), lambda i: (i, 0, 0))],
            out_specs=pl.BlockSpec((BC, T, D), lambda i: (i, 0, 0)),
            out_shape=jax.ShapeDtypeStruct((BCH, T, D), jnp.float32),
        )(gathered.reshape(BCH, T, D))
        outs.append(out_c)
    return jnp.concatenate(outs, axis=0)


# speedup vs baseline: 3.4957x; 3.4957x over previous
import jax
import jax.numpy as jnp
from jax.experimental import pallas as pl

B, T, V, D = 4096, 8, 1024, 256
BB = 256


def _decode_block(msg_ref, cb_ref, out_ref):
    m = msg_ref[...]
    mx = jnp.max(m, axis=-1, keepdims=True)
    iota3 = jax.lax.broadcasted_iota(jnp.int32, (BB, T, V), 2)
    codes = jnp.min(jnp.where(m == mx, iota3, V), axis=-1)
    iota2 = jax.lax.broadcasted_iota(jnp.int32, (BB, V), 1)
    acc = jnp.zeros((BB, D), jnp.float32)
    for t in range(T):
        onehot = (iota2 == codes[:, t : t + 1]).astype(jnp.bfloat16)
        g = jax.lax.dot(onehot, cb_ref[t], preferred_element_type=jnp.float32)
        acc = acc + g
        norm = jnp.sqrt(jnp.sum(acc * acc, axis=-1, keepdims=True))
        out_ref[:, t, :] = acc * (1.0 / jnp.maximum(norm, 1e-12))


@jax.jit
def kernel(message, codebooks):
    cb16 = codebooks.astype(jnp.bfloat16)
    return pl.pallas_call(
        _decode_block,
        grid=(B // BB,),
        in_specs=[
            pl.BlockSpec((BB, T, V), lambda i: (i, 0, 0)),
            pl.BlockSpec((T, V, D), lambda i: (0, 0, 0)),
        ],
        out_specs=pl.BlockSpec((BB, T, D), lambda i: (i, 0, 0)),
        out_shape=jax.ShapeDtypeStruct((B, T, D), jnp.float32),
    )(message, cb16)
